# baseline (speedup 1.0000x reference)
import jax
import jax.numpy as jnp
from jax.experimental import pallas as pl
from jax.experimental.pallas import tpu as pltpu

_BM = 80


def _body(l_ref, x_ref, scale_ref, bias_ref, o_ref, xbf_ref):
    i = pl.program_id(0)

    @pl.when(i == 0)
    def _():
        xbf_ref[...] = x_ref[...].astype(jnp.bfloat16)

    e1 = jnp.dot(
        l_ref[...].astype(jnp.bfloat16),
        xbf_ref[...],
        preferred_element_type=jnp.float32,
    )
    rows = x_ref[pl.ds(i * _BM, _BM), :]
    o_ref[...] = rows - e1 * scale_ref[...] + bias_ref[...]


def kernel(input, l_sym, learnable_diag_1, bias):
    n, f = input.shape
    scale2d = learnable_diag_1.reshape(1, f)
    bias2d = bias.reshape(1, f)
    return pl.pallas_call(
        _body,
        grid=(n // _BM,),
        in_specs=[
            pl.BlockSpec((_BM, n), lambda i: (i, 0)),
            pl.BlockSpec((n, f), lambda i: (0, 0)),
            pl.BlockSpec((1, f), lambda i: (0, 0)),
            pl.BlockSpec((1, f), lambda i: (0, 0)),
        ],
        out_specs=pl.BlockSpec((_BM, f), lambda i: (i, 0)),
        out_shape=jax.ShapeDtypeStruct((n, f), jnp.float32),
        scratch_shapes=[pltpu.VMEM((n, f), jnp.bfloat16)],
    )(l_sym, input, scale2d, bias2d)


# speedup vs baseline: 1.3609x; 1.3609x over previous
import jax
import jax.numpy as jnp
from jax.experimental import pallas as pl
from jax.experimental.pallas import tpu as pltpu

_BM = 400


def _body(l_ref, x_ref, scale_ref, bias_ref, o_ref, xbf_ref):
    i = pl.program_id(0)

    @pl.when(i == 0)
    def _():
        xbf_ref[...] = x_ref[...].astype(jnp.bfloat16)

    e1 = jnp.dot(
        l_ref[...].astype(jnp.bfloat16),
        xbf_ref[...],
        preferred_element_type=jnp.float32,
    )
    rows = x_ref[pl.ds(i * _BM, _BM), :]
    o_ref[...] = rows - e1 * scale_ref[...] + bias_ref[...]


def kernel(input, l_sym, learnable_diag_1, bias):
    n, f = input.shape
    scale2d = learnable_diag_1.reshape(1, f)
    bias2d = bias.reshape(1, f)
    return pl.pallas_call(
        _body,
        grid=(n // _BM,),
        in_specs=[
            pl.BlockSpec((_BM, n), lambda i: (i, 0)),
            pl.BlockSpec((n, f), lambda i: (0, 0)),
            pl.BlockSpec((1, f), lambda i: (0, 0)),
            pl.BlockSpec((1, f), lambda i: (0, 0)),
        ],
        out_specs=pl.BlockSpec((_BM, f), lambda i: (i, 0)),
        out_shape=jax.ShapeDtypeStruct((n, f), jnp.float32),
        scratch_shapes=[pltpu.VMEM((n, f), jnp.bfloat16)],
    )(l_sym, input, scale2d, bias2d)
